# baseline (speedup 1.0000x reference)
import functools

import jax
import jax.numpy as jnp
from jax import lax
from jax.experimental import pallas as pl
from jax.experimental.pallas import tpu as pltpu
from jax.experimental.pallas import tpu_sc as plsc

N = 32768
B = 16
E = 262144
D = 128
H = 64
C = 10

SC_NC = 2
SC_NS = 16
SC_L = 16
SC_NW = SC_NC * SC_NS
EPW = E // SC_NW


NBLK = 8192
N_STEPS1 = N // NBLK


def _fil_kernel(x_ref, wf_ref, bf_ref, lo_ref, hi_ref,
                filt_ref, h0_ref, acc_ref):
    pid = pl.program_id(0)
    filt_row = lax.dot_general(wf_ref[...], x_ref[...],
                               (((1,), (1,)), ((), ())),
                               preferred_element_type=jnp.float32)
    filt_row = filt_row + bf_ref[0, 0]
    filt_ref[0] = filt_row
    ones_row = jnp.ones((1, NBLK), jnp.float32)
    lof = jnp.dot(lo_ref[...], ones_row,
                  preferred_element_type=jnp.float32)
    hif = jnp.dot(hi_ref[...], ones_row,
                  preferred_element_type=jnp.float32)
    ids = (jnp.float32(pid * NBLK)
           + lax.broadcasted_iota(jnp.int32, (B, NBLK), 1).astype(jnp.float32))
    onehot = (ids >= lof) & (ids < hif)
    filtb = jnp.broadcast_to(filt_row, (B, NBLK))
    part = jnp.min(jnp.where(onehot, filtb, jnp.inf), axis=1, keepdims=True)

    @pl.when(pid == 0)
    def _():
        acc_ref[...] = part

    @pl.when(pid > 0)
    def _():
        acc_ref[...] = jnp.minimum(acc_ref[...], part)

    @pl.when(pid == N_STEPS1 - 1)
    def _():
        h0_ref[...] = acc_ref[...]


def _run_fil(x, w_fil, b_fil, lo, hi):
    return pl.pallas_call(
        _fil_kernel,
        grid=(N_STEPS1,),
        in_specs=[
            pl.BlockSpec((NBLK, D), lambda i: (i, 0)),
            pl.BlockSpec((1, D), lambda i: (0, 0)),
            pl.BlockSpec((1, 1), lambda i: (0, 0)),
            pl.BlockSpec((B, 1), lambda i: (0, 0)),
            pl.BlockSpec((B, 1), lambda i: (0, 0)),
        ],
        out_specs=[
            pl.BlockSpec((1, 1, NBLK), lambda i: (i, 0, 0)),
            pl.BlockSpec((B, 1), lambda i: (0, 0)),
        ],
        out_shape=[
            jax.ShapeDtypeStruct((N_STEPS1, 1, NBLK), jnp.float32),
            jax.ShapeDtypeStruct((B, 1), jnp.float32),
        ],
        scratch_shapes=[pltpu.VMEM((B, 1), jnp.float32)],
    )(x, w_fil, b_fil, lo, hi)


def _sc_gather_body(filt_hbm, ei_hbm, pairs_hbm,
                    filt_v, src_v, dst_v, birth_v, death_v,
                    sem_f, sem_s, sem_d):
    wid = lax.axis_index("s") * SC_NC + lax.axis_index("c")
    base = wid * EPW
    cf = pltpu.async_copy(filt_hbm, filt_v, sem_f)
    cs = pltpu.async_copy(ei_hbm.at[0, pl.ds(base, EPW)], src_v, sem_s)
    cd = pltpu.async_copy(ei_hbm.at[1, pl.ds(base, EPW)], dst_v, sem_d)
    cf.wait()
    cs.wait()
    cd.wait()

    def body(i):
        off = i * SC_L
        su = src_v[pl.ds(off, SC_L)]
        sv = dst_v[pl.ds(off, SC_L)]
        fu = plsc.load_gather(filt_v, [su])
        fv = plsc.load_gather(filt_v, [sv])
        birth_v[pl.ds(off, SC_L)] = jnp.minimum(fu, fv)
        death_v[pl.ds(off, SC_L)] = jnp.maximum(fu, fv)

    plsc.parallel_loop(0, EPW // SC_L, 1, unroll=16)(body)
    cb = pltpu.async_copy(birth_v, pairs_hbm.at[0, pl.ds(base, EPW)], sem_s)
    cdd = pltpu.async_copy(death_v, pairs_hbm.at[1, pl.ds(base, EPW)], sem_d)
    cb.wait()
    cdd.wait()


def _run_sc_gather(filt, edge_index):
    mesh = plsc.VectorSubcoreMesh(core_axis_name="c", subcore_axis_name="s")
    k = pl.kernel(
        _sc_gather_body,
        out_type=jax.ShapeDtypeStruct((2, E), jnp.float32),
        mesh=mesh,
        compiler_params=pltpu.CompilerParams(needs_layout_passes=False),
        scratch_types=[
            pltpu.VMEM((N,), jnp.float32),
            pltpu.VMEM((EPW,), jnp.int32),
            pltpu.VMEM((EPW,), jnp.int32),
            pltpu.VMEM((EPW,), jnp.float32),
            pltpu.VMEM((EPW,), jnp.float32),
            pltpu.SemaphoreType.DMA,
            pltpu.SemaphoreType.DMA,
            pltpu.SemaphoreType.DMA,
        ],
    )
    return k(filt, edge_index)


EBLK = 32768
N_STEPS3 = E // EBLK


def _seg_kernel(pairs_ref, seg_ref, wpb_ref, h0_ref, wh_ref, bh_ref,
                out_ref, accf_ref, acch1_ref):
    pid = pl.program_id(0)
    bd = pairs_ref[...]
    ones = jnp.ones((1, EBLK), jnp.float32)
    b3 = jnp.concatenate([bd, ones], axis=0)
    feat_t = jnp.maximum(
        jnp.dot(wpb_ref[...], b3, preferred_element_type=jnp.float32), 0.0)
    segrow = seg_ref[0]
    onehot = (jnp.broadcast_to(segrow, (B, EBLK))
              == lax.broadcasted_iota(jnp.int32, (B, EBLK), 0))
    part = lax.dot_general(onehot.astype(jnp.float32), feat_t,
                           (((1,), (1,)), ((), ())),
                           preferred_element_type=jnp.float32)
    ddb = jnp.broadcast_to(bd[1:2, :], (B, EBLK))
    h1p = jnp.max(jnp.where(onehot, ddb, -jnp.inf), axis=1, keepdims=True)

    @pl.when(pid == 0)
    def _():
        accf_ref[...] = part
        acch1_ref[...] = h1p

    @pl.when(pid > 0)
    def _():
        accf_ref[...] = accf_ref[...] + part
        acch1_ref[...] = jnp.maximum(acch1_ref[...], h1p)

    @pl.when(pid == N_STEPS3 - 1)
    def _():
        gf = accf_ref[...]
        h0c = h0_ref[...]
        h1c = acch1_ref[...]
        y = jnp.dot(gf, wh_ref[0:H, :], preferred_element_type=jnp.float32)
        y = y + h0c * wh_ref[H:H + 1, :] + h1c * wh_ref[H + 1:H + 2, :]
        out_ref[...] = y + bh_ref[...]


def _run_seg(pairs, seg2d, wpb, h0, w_head, b_head):
    return pl.pallas_call(
        _seg_kernel,
        grid=(N_STEPS3,),
        in_specs=[
            pl.BlockSpec((2, EBLK), lambda i: (0, i)),
            pl.BlockSpec((1, 1, EBLK), lambda i: (i, 0, 0)),
            pl.BlockSpec((H, 3), lambda i: (0, 0)),
            pl.BlockSpec((B, 1), lambda i: (0, 0)),
            pl.BlockSpec((H + 2, C), lambda i: (0, 0)),
            pl.BlockSpec((1, C), lambda i: (0, 0)),
        ],
        out_specs=pl.BlockSpec((B, C), lambda i: (0, 0)),
        out_shape=jax.ShapeDtypeStruct((B, C), jnp.float32),
        scratch_shapes=[
            pltpu.VMEM((B, H), jnp.float32),
            pltpu.VMEM((B, 1), jnp.float32),
        ],
    )(pairs, seg2d, wpb, h0, w_head, b_head)


@jax.jit
def kernel(x, cu_seqlens, edge_index, edge_segment_ids,
           W_fil, b_fil, W_pair, b_pair, W_head, b_head):
    lo = cu_seqlens[:B].astype(jnp.float32).reshape(B, 1)
    hi = cu_seqlens[1:B + 1].astype(jnp.float32).reshape(B, 1)
    filt3d, h0 = _run_fil(x, W_fil.reshape(1, D), b_fil.reshape(1, 1), lo, hi)
    filt = filt3d.reshape(N)
    pairs = _run_sc_gather(filt, edge_index)
    wpb = jnp.concatenate([W_pair.T, b_pair.reshape(H, 1)], axis=1)
    y = _run_seg(pairs, edge_segment_ids.reshape(N_STEPS3, 1, EBLK),
                 wpb, h0, W_head, b_head.reshape(1, C))
    return y


# speedup vs baseline: 1.0035x; 1.0035x over previous
import jax
import jax.numpy as jnp
from jax import lax
from jax.experimental import pallas as pl
from jax.experimental.pallas import tpu as pltpu
from jax.experimental.pallas import tpu_sc as plsc

N = 32768
B = 16
E = 262144
D = 128
H = 64
C = 10

SC_NC = 2
SC_NS = 16
SC_L = 16
SC_NW = SC_NC * SC_NS
EPW = E // SC_NW


NBLK = 8192
N_STEPS1 = N // NBLK


def _fil_kernel(x_ref, wf_ref, bf_ref, lo_ref, hi_ref,
                filt_ref, h0_ref, acc_ref):
    pid = pl.program_id(0)
    filt_row = lax.dot_general(wf_ref[...], x_ref[...],
                               (((1,), (1,)), ((), ())),
                               preferred_element_type=jnp.float32)
    filt_row = filt_row + bf_ref[0, 0]
    filt_ref[0] = filt_row
    ones_row = jnp.ones((1, NBLK), jnp.float32)
    lof = jnp.dot(lo_ref[...], ones_row,
                  preferred_element_type=jnp.float32)
    hif = jnp.dot(hi_ref[...], ones_row,
                  preferred_element_type=jnp.float32)
    ids = (jnp.float32(pid * NBLK)
           + lax.broadcasted_iota(jnp.int32, (B, NBLK), 1).astype(jnp.float32))
    onehot = (ids >= lof) & (ids < hif)
    filtb = jnp.broadcast_to(filt_row, (B, NBLK))
    part = jnp.min(jnp.where(onehot, filtb, jnp.inf), axis=1, keepdims=True)

    @pl.when(pid == 0)
    def _():
        acc_ref[...] = part

    @pl.when(pid > 0)
    def _():
        acc_ref[...] = jnp.minimum(acc_ref[...], part)

    @pl.when(pid == N_STEPS1 - 1)
    def _():
        h0_ref[...] = acc_ref[...]


def _run_fil(x, w_fil, b_fil, lo, hi):
    return pl.pallas_call(
        _fil_kernel,
        grid=(N_STEPS1,),
        in_specs=[
            pl.BlockSpec((NBLK, D), lambda i: (i, 0)),
            pl.BlockSpec((1, D), lambda i: (0, 0)),
            pl.BlockSpec((1, 1), lambda i: (0, 0)),
            pl.BlockSpec((B, 1), lambda i: (0, 0)),
            pl.BlockSpec((B, 1), lambda i: (0, 0)),
        ],
        out_specs=[
            pl.BlockSpec((1, 1, NBLK), lambda i: (i, 0, 0)),
            pl.BlockSpec((B, 1), lambda i: (0, 0)),
        ],
        out_shape=[
            jax.ShapeDtypeStruct((N_STEPS1, 1, NBLK), jnp.float32),
            jax.ShapeDtypeStruct((B, 1), jnp.float32),
        ],
        scratch_shapes=[pltpu.VMEM((B, 1), jnp.float32)],
    )(x, w_fil, b_fil, lo, hi)


def _sc_gather_body(filt_hbm, ei_hbm, pairs_hbm,
                    filt_v, src_v, dst_v, birth_v, death_v,
                    sem_f, sem_s, sem_d):
    wid = lax.axis_index("s") * SC_NC + lax.axis_index("c")
    base = wid * EPW
    cf = pltpu.async_copy(filt_hbm, filt_v, sem_f)
    cs = pltpu.async_copy(ei_hbm.at[0, pl.ds(base, EPW)], src_v, sem_s)
    cd = pltpu.async_copy(ei_hbm.at[1, pl.ds(base, EPW)], dst_v, sem_d)
    cf.wait()
    cs.wait()
    cd.wait()

    def body(i):
        off = i * SC_L
        su = src_v[pl.ds(off, SC_L)]
        sv = dst_v[pl.ds(off, SC_L)]
        fu = plsc.load_gather(filt_v, [su])
        fv = plsc.load_gather(filt_v, [sv])
        birth_v[pl.ds(off, SC_L)] = jnp.minimum(fu, fv)
        death_v[pl.ds(off, SC_L)] = jnp.maximum(fu, fv)

    plsc.parallel_loop(0, EPW // SC_L, 1, unroll=8)(body)
    cb = pltpu.async_copy(birth_v, pairs_hbm.at[0, pl.ds(base, EPW)], sem_s)
    cdd = pltpu.async_copy(death_v, pairs_hbm.at[1, pl.ds(base, EPW)], sem_d)
    cb.wait()
    cdd.wait()


def _run_sc_gather(filt, edge_index):
    mesh = plsc.VectorSubcoreMesh(core_axis_name="c", subcore_axis_name="s")
    k = pl.kernel(
        _sc_gather_body,
        out_type=jax.ShapeDtypeStruct((2, E), jnp.float32),
        mesh=mesh,
        compiler_params=pltpu.CompilerParams(needs_layout_passes=False),
        scratch_types=[
            pltpu.VMEM((N,), jnp.float32),
            pltpu.VMEM((EPW,), jnp.int32),
            pltpu.VMEM((EPW,), jnp.int32),
            pltpu.VMEM((EPW,), jnp.float32),
            pltpu.VMEM((EPW,), jnp.float32),
            pltpu.SemaphoreType.DMA,
            pltpu.SemaphoreType.DMA,
            pltpu.SemaphoreType.DMA,
        ],
    )
    return k(filt, edge_index)


EBLK = 32768
N_STEPS3 = E // EBLK


def _seg_kernel(pairs_ref, seg_ref, wpb_ref, h0_ref, wh_ref, bh_ref,
                out_ref, accf_ref, acch1_ref):
    pid = pl.program_id(0)
    bd = pairs_ref[...]
    ones = jnp.ones((1, EBLK), jnp.float32)
    b3 = jnp.concatenate([bd, ones], axis=0)
    feat_t = jnp.maximum(
        jnp.dot(wpb_ref[...], b3, preferred_element_type=jnp.float32), 0.0)
    segrow = seg_ref[0]
    onehot = (jnp.broadcast_to(segrow, (B, EBLK))
              == lax.broadcasted_iota(jnp.int32, (B, EBLK), 0))
    part = lax.dot_general(onehot.astype(jnp.float32), feat_t,
                           (((1,), (1,)), ((), ())),
                           preferred_element_type=jnp.float32)
    ddb = jnp.broadcast_to(bd[1:2, :], (B, EBLK))
    h1p = jnp.max(jnp.where(onehot, ddb, -jnp.inf), axis=1, keepdims=True)

    @pl.when(pid == 0)
    def _():
        accf_ref[...] = part
        acch1_ref[...] = h1p

    @pl.when(pid > 0)
    def _():
        accf_ref[...] = accf_ref[...] + part
        acch1_ref[...] = jnp.maximum(acch1_ref[...], h1p)

    @pl.when(pid == N_STEPS3 - 1)
    def _():
        gf = accf_ref[...]
        h0c = h0_ref[...]
        h1c = acch1_ref[...]
        y = jnp.dot(gf, wh_ref[0:H, :], preferred_element_type=jnp.float32)
        y = y + h0c * wh_ref[H:H + 1, :] + h1c * wh_ref[H + 1:H + 2, :]
        out_ref[...] = y + bh_ref[...]


def _run_seg(pairs, seg2d, wpb, h0, w_head, b_head):
    return pl.pallas_call(
        _seg_kernel,
        grid=(N_STEPS3,),
        in_specs=[
            pl.BlockSpec((2, EBLK), lambda i: (0, i)),
            pl.BlockSpec((1, 1, EBLK), lambda i: (i, 0, 0)),
            pl.BlockSpec((H, 3), lambda i: (0, 0)),
            pl.BlockSpec((B, 1), lambda i: (0, 0)),
            pl.BlockSpec((H + 2, C), lambda i: (0, 0)),
            pl.BlockSpec((1, C), lambda i: (0, 0)),
        ],
        out_specs=pl.BlockSpec((B, C), lambda i: (0, 0)),
        out_shape=jax.ShapeDtypeStruct((B, C), jnp.float32),
        scratch_shapes=[
            pltpu.VMEM((B, H), jnp.float32),
            pltpu.VMEM((B, 1), jnp.float32),
        ],
    )(pairs, seg2d, wpb, h0, w_head, b_head)


@jax.jit
def kernel(x, cu_seqlens, edge_index, edge_segment_ids,
           W_fil, b_fil, W_pair, b_pair, W_head, b_head):
    lo = cu_seqlens[:B].astype(jnp.float32).reshape(B, 1)
    hi = cu_seqlens[1:B + 1].astype(jnp.float32).reshape(B, 1)
    filt3d, h0 = _run_fil(x, W_fil.reshape(1, D), b_fil.reshape(1, 1), lo, hi)
    filt = filt3d.reshape(N)
    pairs = _run_sc_gather(filt, edge_index)
    wpb = jnp.concatenate([W_pair.T, b_pair.reshape(H, 1)], axis=1)
    y = _run_seg(pairs, edge_segment_ids.reshape(N_STEPS3, 1, EBLK),
                 wpb, h0, W_head, b_head.reshape(1, C))
    return y
